# baseline (speedup 1.0000x reference)
import functools

import jax
import jax.numpy as jnp
from jax import lax
from jax.experimental import pallas as pl
from jax.experimental.pallas import tpu as pltpu
from jax.experimental.pallas import tpu_sc as plsc

NODE_SIZE = 100000
EMBED_SIZE = 128
BATCH = 16384

L = 16
NW = 32
B_PER_W = BATCH // NW
CHUNK = 128
NCHUNK = B_PER_W // CHUNK
UNROLL = 2

_LOG1P_COEF = (
    1.446112683e-03, -1.027147447e-02, 3.423174471e-02, -7.301764925e-02,
    1.166124657e-01, -1.571737904e-01, 1.976391457e-01, -2.496172750e-01,
    3.332960370e-01, -4.999980978e-01, 9.999999616e-01, 0.0,
)


def _log1p_poly(u):
    acc = jnp.full((L,), _LOG1P_COEF[0], dtype=jnp.float32)
    for c in _LOG1P_COEF[1:]:
        acc = acc * u + c
    return acc


_BITREV16 = (0, 8, 4, 12, 2, 10, 6, 14, 1, 9, 5, 13, 3, 11, 7, 15)


def _sc_kernel(a_hbm, b_hbm, sign_hbm, emb_hbm, ctx_hbm, out_hbm,
               idx_a, idx_b, sign_v, arows0, brows0, arows1, brows1, dots,
               sem_a0, sem_b0, sem_a1, sem_b1):
    wid = lax.axis_index("s") * 2 + lax.axis_index("c")
    row0 = wid * NCHUNK
    base = wid * B_PER_W

    pltpu.sync_copy(a_hbm.at[pl.ds(row0, NCHUNK)], idx_a)
    pltpu.sync_copy(b_hbm.at[pl.ds(row0, NCHUNK)], idx_b)
    pltpu.sync_copy(sign_hbm.at[pl.ds(base, B_PER_W)], sign_v)

    lane = lax.iota(jnp.int32, L)
    perms = {h: lane ^ h for h in (8, 4, 2, 1)}
    masks = {h: (lane & h) == 0 for h in (8, 4, 2, 1)}
    bitrev = (((lane & 1) << 3) | ((lane & 2) << 1)
              | ((lane & 4) >> 1) | ((lane & 8) >> 3))

    def merge(x0, x1, h):
        m, p = masks[h], perms[h]
        return (jnp.where(m, x0, jnp.take(x1, p))
                + jnp.where(m, jnp.take(x0, p), x1))

    bufs = ((arows0, brows0, sem_a0, sem_b0), (arows1, brows1, sem_a1, sem_b1))

    def start(c):
        arows, brows, sem_a, sem_b = bufs[c % 2]
        cp_a = pltpu.async_copy(emb_hbm.at[idx_a.at[c]], arows, sem_a)
        cp_b = pltpu.async_copy(ctx_hbm.at[idx_b.at[c]], brows, sem_b)
        return cp_a, cp_b

    inflight = start(0)
    for c in range(NCHUNK):
        arows, brows, _, _ = bufs[c % 2]
        cp_a, cp_b = inflight
        cp_a.wait()
        cp_b.wait()
        if c + 1 < NCHUNK:
            inflight = start(c + 1)

        @plsc.parallel_loop(0, CHUNK // L, unroll=UNROLL)
        def _dot_body(g, c=c, arows=arows, brows=brows):
            gbase = g * L
            vecs = []
            for t in range(L):
                r = gbase + t
                acc0 = (arows[r, pl.ds(0, L)] * brows[r, pl.ds(0, L)])
                acc1 = (arows[r, pl.ds(L, L)] * brows[r, pl.ds(L, L)])
                for j in range(2, EMBED_SIZE // L, 2):
                    acc0 = acc0 + arows[r, pl.ds(j * L, L)] * brows[r, pl.ds(j * L, L)]
                    acc1 = acc1 + arows[r, pl.ds((j + 1) * L, L)] * brows[r, pl.ds((j + 1) * L, L)]
                vecs.append(acc0 + acc1)
            for h in (8, 4, 2, 1):
                vecs = [merge(vecs[2 * i], vecs[2 * i + 1], h)
                        for i in range(len(vecs) // 2)]
            dot16 = jnp.take(vecs[0], bitrev)
            off = pl.multiple_of(c * CHUNK + gbase, L)
            t = sign_v[pl.ds(off, L)] * dot16
            u = jnp.exp(-jnp.abs(t))
            dots[pl.ds(off, L)] = jnp.maximum(-t, 0.0) + _log1p_poly(u)

    pltpu.sync_copy(dots, out_hbm.at[pl.ds(base, B_PER_W)])


@jax.jit
def _run(a2d, b2d, sign, emb_table, ctx_table):
    mesh = plsc.VectorSubcoreMesh(core_axis_name="c", subcore_axis_name="s")
    f = pl.kernel(
        _sc_kernel,
        mesh=mesh,
        compiler_params=pltpu.CompilerParams(needs_layout_passes=False),
        out_type=jax.ShapeDtypeStruct((BATCH,), jnp.float32),
        scratch_types=[
            pltpu.VMEM((NCHUNK, CHUNK), jnp.int32),
            pltpu.VMEM((NCHUNK, CHUNK), jnp.int32),
            pltpu.VMEM((B_PER_W,), jnp.float32),
            pltpu.VMEM((CHUNK, EMBED_SIZE), jnp.float32),
            pltpu.VMEM((CHUNK, EMBED_SIZE), jnp.float32),
            pltpu.VMEM((CHUNK, EMBED_SIZE), jnp.float32),
            pltpu.VMEM((CHUNK, EMBED_SIZE), jnp.float32),
            pltpu.VMEM((B_PER_W,), jnp.float32),
            pltpu.SemaphoreType.DMA,
            pltpu.SemaphoreType.DMA,
            pltpu.SemaphoreType.DMA,
            pltpu.SemaphoreType.DMA,
        ],
    )
    return f(a2d, b2d, sign, emb_table, ctx_table)


def kernel(a, b, sign, emb_table, ctx_table):
    a2d = a.reshape(BATCH // CHUNK, CHUNK)
    b2d = b.reshape(BATCH // CHUNK, CHUNK)
    return _run(a2d, b2d, sign, emb_table, ctx_table)


# speedup vs baseline: 1.3598x; 1.3598x over previous
import functools

import jax
import jax.numpy as jnp
from jax import lax
from jax.experimental import pallas as pl
from jax.experimental.pallas import tpu as pltpu
from jax.experimental.pallas import tpu_sc as plsc

NODE_SIZE = 100000
EMBED_SIZE = 128
BATCH = 16384

L = 16
NW = 32
B_PER_W = BATCH // NW
CHUNK = 128
NCHUNK = B_PER_W // CHUNK
UNROLL = 1

_LOG1P_COEF = (
    1.446112683e-03, -1.027147447e-02, 3.423174471e-02, -7.301764925e-02,
    1.166124657e-01, -1.571737904e-01, 1.976391457e-01, -2.496172750e-01,
    3.332960370e-01, -4.999980978e-01, 9.999999616e-01, 0.0,
)


def _log1p_poly(u):
    acc = jnp.full((L,), _LOG1P_COEF[0], dtype=jnp.float32)
    for c in _LOG1P_COEF[1:]:
        acc = acc * u + c
    return acc


_BITREV16 = (0, 8, 4, 12, 2, 10, 6, 14, 1, 9, 5, 13, 3, 11, 7, 15)


def _sc_kernel(a_hbm, b_hbm, sign_hbm, emb_hbm, ctx_hbm, out_hbm,
               idx_a, idx_b, sign_v, arows0, brows0, arows1, brows1, dots,
               sem_a0, sem_b0, sem_a1, sem_b1):
    wid = lax.axis_index("s") * 2 + lax.axis_index("c")
    row0 = wid * NCHUNK
    base = wid * B_PER_W

    pltpu.sync_copy(a_hbm.at[pl.ds(row0, NCHUNK)], idx_a)
    pltpu.sync_copy(b_hbm.at[pl.ds(row0, NCHUNK)], idx_b)
    pltpu.sync_copy(sign_hbm.at[pl.ds(base, B_PER_W)], sign_v)

    lane = lax.iota(jnp.int32, L)
    perms = {h: lane ^ h for h in (8, 4, 2, 1)}
    masks = {h: (lane & h) == 0 for h in (8, 4, 2, 1)}
    mask_even = (lane & 1) == 0
    pair = lane >> 1
    rev3 = ((pair & 1) << 2) | (pair & 2) | ((pair & 4) >> 2)

    def merge(x0, x1, h):
        m, p = masks[h], perms[h]
        return (jnp.where(m, x0, jnp.take(x1, p))
                + jnp.where(m, jnp.take(x0, p), x1))

    bufs = ((arows0, brows0, sem_a0, sem_b0), (arows1, brows1, sem_a1, sem_b1))

    def start(c):
        arows, brows, sem_a, sem_b = bufs[c % 2]
        cp_a = pltpu.async_copy(emb_hbm.at[idx_a.at[c]], arows, sem_a)
        cp_b = pltpu.async_copy(ctx_hbm.at[idx_b.at[c]], brows, sem_b)
        return cp_a, cp_b

    inflight = start(0)
    for c in range(NCHUNK):
        arows, brows, _, _ = bufs[c % 2]
        cp_a, cp_b = inflight
        cp_a.wait()
        cp_b.wait()
        if c + 1 < NCHUNK:
            inflight = start(c + 1)

        @plsc.parallel_loop(0, CHUNK // 8, unroll=UNROLL)
        def _dot_body(g, c=c, arows=arows, brows=brows):
            gbase = g * 8
            pending = {}
            for t in range(8):
                r = gbase + t
                acc0 = (arows[r, pl.ds(0, L)] * brows[r, pl.ds(0, L)])
                acc1 = (arows[r, pl.ds(L, L)] * brows[r, pl.ds(L, L)])
                for j in range(2, EMBED_SIZE // L, 2):
                    acc0 = acc0 + arows[r, pl.ds(j * L, L)] * brows[r, pl.ds(j * L, L)]
                    acc1 = acc1 + arows[r, pl.ds((j + 1) * L, L)] * brows[r, pl.ds((j + 1) * L, L)]
                v = acc0 + acc1
                h = 8
                while h in pending:
                    v = merge(pending.pop(h), v, h)
                    h //= 2
                pending[h] = v
            v2 = pending[1]
            dotv = v2 + jnp.take(v2, lane ^ 1)
            rows16 = gbase + c * CHUNK + rev3
            t = plsc.load_gather(sign_v, [rows16]) * dotv
            u = jnp.exp(-jnp.abs(t))
            loss = jnp.maximum(-t, 0.0) + _log1p_poly(u)
            plsc.store_scatter(dots, [rows16], loss, mask=mask_even)

    pltpu.sync_copy(dots, out_hbm.at[pl.ds(base, B_PER_W)])


@jax.jit
def _run(a2d, b2d, sign, emb_table, ctx_table):
    mesh = plsc.VectorSubcoreMesh(core_axis_name="c", subcore_axis_name="s")
    f = pl.kernel(
        _sc_kernel,
        mesh=mesh,
        compiler_params=pltpu.CompilerParams(needs_layout_passes=False),
        out_type=jax.ShapeDtypeStruct((BATCH,), jnp.float32),
        scratch_types=[
            pltpu.VMEM((NCHUNK, CHUNK), jnp.int32),
            pltpu.VMEM((NCHUNK, CHUNK), jnp.int32),
            pltpu.VMEM((B_PER_W,), jnp.float32),
            pltpu.VMEM((CHUNK, EMBED_SIZE), jnp.float32),
            pltpu.VMEM((CHUNK, EMBED_SIZE), jnp.float32),
            pltpu.VMEM((CHUNK, EMBED_SIZE), jnp.float32),
            pltpu.VMEM((CHUNK, EMBED_SIZE), jnp.float32),
            pltpu.VMEM((B_PER_W,), jnp.float32),
            pltpu.SemaphoreType.DMA,
            pltpu.SemaphoreType.DMA,
            pltpu.SemaphoreType.DMA,
            pltpu.SemaphoreType.DMA,
        ],
    )
    return f(a2d, b2d, sign, emb_table, ctx_table)


def kernel(a, b, sign, emb_table, ctx_table):
    a2d = a.reshape(BATCH // CHUNK, CHUNK)
    b2d = b.reshape(BATCH // CHUNK, CHUNK)
    return _run(a2d, b2d, sign, emb_table, ctx_table)


# speedup vs baseline: 1.5140x; 1.1134x over previous
import functools

import jax
import jax.numpy as jnp
from jax import lax
from jax.experimental import pallas as pl
from jax.experimental.pallas import tpu as pltpu
from jax.experimental.pallas import tpu_sc as plsc

NODE_SIZE = 100000
EMBED_SIZE = 128
BATCH = 16384

L = 16
NW = 32
B_PER_W = BATCH // NW
CHUNK = 128
NCHUNK = B_PER_W // CHUNK
UNROLL = 4

_LOG1P_COEF = (
    1.446112683e-03, -1.027147447e-02, 3.423174471e-02, -7.301764925e-02,
    1.166124657e-01, -1.571737904e-01, 1.976391457e-01, -2.496172750e-01,
    3.332960370e-01, -4.999980978e-01, 9.999999616e-01, 0.0,
)


def _log1p_poly(u):
    acc = jnp.full((L,), _LOG1P_COEF[0], dtype=jnp.float32)
    for c in _LOG1P_COEF[1:]:
        acc = acc * u + c
    return acc


_BITREV16 = (0, 8, 4, 12, 2, 10, 6, 14, 1, 9, 5, 13, 3, 11, 7, 15)


def _sc_kernel(a_hbm, b_hbm, sign_hbm, emb_hbm, ctx_hbm, out_hbm,
               idx_a, idx_b, sign_v, arows0, brows0, arows1, brows1, dots,
               sem_a0, sem_b0, sem_a1, sem_b1):
    wid = lax.axis_index("s") * 2 + lax.axis_index("c")
    row0 = wid * NCHUNK
    base = wid * B_PER_W

    pltpu.sync_copy(a_hbm.at[pl.ds(row0, NCHUNK)], idx_a)
    pltpu.sync_copy(b_hbm.at[pl.ds(row0, NCHUNK)], idx_b)
    pltpu.sync_copy(sign_hbm.at[pl.ds(base, B_PER_W)], sign_v)

    lane = lax.iota(jnp.int32, L)
    perms = {h: lane ^ h for h in (8, 4, 2, 1)}
    lane_masks = [lane == t for t in range(L)]

    bufs = ((arows0, brows0, sem_a0, sem_b0), (arows1, brows1, sem_a1, sem_b1))

    def start(c):
        arows, brows, sem_a, sem_b = bufs[c % 2]
        cp_a = pltpu.async_copy(emb_hbm.at[idx_a.at[c]], arows, sem_a)
        cp_b = pltpu.async_copy(ctx_hbm.at[idx_b.at[c]], brows, sem_b)
        return cp_a, cp_b

    inflight = start(0)
    for c in range(NCHUNK):
        arows, brows, _, _ = bufs[c % 2]
        cp_a, cp_b = inflight
        cp_a.wait()
        cp_b.wait()
        if c + 1 < NCHUNK:
            inflight = start(c + 1)

        @plsc.parallel_loop(0, CHUNK, unroll=UNROLL)
        def _dot_body(r, c=c, arows=arows, brows=brows):
            acc0 = (arows[r, pl.ds(0, L)] * brows[r, pl.ds(0, L)])
            acc1 = (arows[r, pl.ds(L, L)] * brows[r, pl.ds(L, L)])
            for j in range(2, EMBED_SIZE // L, 2):
                acc0 = acc0 + arows[r, pl.ds(j * L, L)] * brows[r, pl.ds(j * L, L)]
                acc1 = acc1 + arows[r, pl.ds((j + 1) * L, L)] * brows[r, pl.ds((j + 1) * L, L)]
            v = acc0 + acc1
            for h in (8, 4, 2, 1):
                v = v + jnp.take(v, perms[h])
            plsc.store_scatter(dots, [jnp.full((L,), c * CHUNK + r, jnp.int32)],
                               v, mask=lane_masks[0])

    @plsc.parallel_loop(0, B_PER_W // L, unroll=4)
    def _loss_body(i):
        off = pl.multiple_of(i * L, L)
        t = sign_v[pl.ds(off, L)] * dots[pl.ds(off, L)]
        u = jnp.exp(-jnp.abs(t))
        dots[pl.ds(off, L)] = jnp.maximum(-t, 0.0) + _log1p_poly(u)

    pltpu.sync_copy(dots, out_hbm.at[pl.ds(base, B_PER_W)])


@jax.jit
def _run(a2d, b2d, sign, emb_table, ctx_table):
    mesh = plsc.VectorSubcoreMesh(core_axis_name="c", subcore_axis_name="s")
    f = pl.kernel(
        _sc_kernel,
        mesh=mesh,
        compiler_params=pltpu.CompilerParams(needs_layout_passes=False),
        out_type=jax.ShapeDtypeStruct((BATCH,), jnp.float32),
        scratch_types=[
            pltpu.VMEM((NCHUNK, CHUNK), jnp.int32),
            pltpu.VMEM((NCHUNK, CHUNK), jnp.int32),
            pltpu.VMEM((B_PER_W,), jnp.float32),
            pltpu.VMEM((CHUNK, EMBED_SIZE), jnp.float32),
            pltpu.VMEM((CHUNK, EMBED_SIZE), jnp.float32),
            pltpu.VMEM((CHUNK, EMBED_SIZE), jnp.float32),
            pltpu.VMEM((CHUNK, EMBED_SIZE), jnp.float32),
            pltpu.VMEM((B_PER_W,), jnp.float32),
            pltpu.SemaphoreType.DMA,
            pltpu.SemaphoreType.DMA,
            pltpu.SemaphoreType.DMA,
            pltpu.SemaphoreType.DMA,
        ],
    )
    return f(a2d, b2d, sign, emb_table, ctx_table)


def kernel(a, b, sign, emb_table, ctx_table):
    a2d = a.reshape(BATCH // CHUNK, CHUNK)
    b2d = b.reshape(BATCH // CHUNK, CHUNK)
    return _run(a2d, b2d, sign, emb_table, ctx_table)


# speedup vs baseline: 1.5705x; 1.0373x over previous
import jax
import jax.numpy as jnp
from jax import lax
from jax.experimental import pallas as pl
from jax.experimental.pallas import tpu as pltpu
from jax.experimental.pallas import tpu_sc as plsc

NODE_SIZE = 100000
EMBED_SIZE = 128
BATCH = 16384

L = 16
NW = 32
B_PER_W = BATCH // NW
CHUNK = 64
NCHUNK = B_PER_W // CHUNK
NBUF = 3
UNROLL = 4

_LOG1P_COEF = (
    1.446112683e-03, -1.027147447e-02, 3.423174471e-02, -7.301764925e-02,
    1.166124657e-01, -1.571737904e-01, 1.976391457e-01, -2.496172750e-01,
    3.332960370e-01, -4.999980978e-01, 9.999999616e-01, 0.0,
)


def _log1p_poly(u):
    acc = jnp.full((L,), _LOG1P_COEF[0], dtype=jnp.float32)
    for c in _LOG1P_COEF[1:]:
        acc = acc * u + c
    return acc


def _sc_kernel(a_hbm, b_hbm, sign_hbm, emb_hbm, ctx_hbm, out_hbm,
               idx_a, idx_b, sign_v, dots, rows, sems, sem_i, sem_s):
    wid = lax.axis_index("s") * 2 + lax.axis_index("c")
    row0 = wid * NCHUNK
    base = wid * B_PER_W

    cp_ia = pltpu.async_copy(a_hbm.at[pl.ds(row0, NCHUNK)], idx_a, sem_i)
    cp_ib = pltpu.async_copy(b_hbm.at[pl.ds(row0, NCHUNK)], idx_b, sem_i)
    cp_sg = pltpu.async_copy(sign_hbm.at[pl.ds(base, B_PER_W)], sign_v, sem_s)
    cp_ia.wait()
    cp_ib.wait()

    lane = lax.iota(jnp.int32, L)
    perms = {h: lane ^ h for h in (8, 4, 2, 1)}
    mask0 = lane == 0

    def start(c):
        s = c % NBUF
        cp_a = pltpu.async_copy(emb_hbm.at[idx_a.at[c]], rows[2 * s], sems[2 * s])
        cp_b = pltpu.async_copy(ctx_hbm.at[idx_b.at[c]], rows[2 * s + 1], sems[2 * s + 1])
        return cp_a, cp_b

    inflight = [start(0), start(1)]
    for c in range(NCHUNK):
        s = c % NBUF
        arows, brows = rows[2 * s], rows[2 * s + 1]
        cp_a, cp_b = inflight.pop(0)
        cp_a.wait()
        cp_b.wait()
        if c + 2 < NCHUNK:
            inflight.append(start(c + 2))

        @plsc.parallel_loop(0, CHUNK, unroll=UNROLL)
        def _dot_body(r, c=c, arows=arows, brows=brows):
            acc0 = (arows[r, pl.ds(0, L)] * brows[r, pl.ds(0, L)])
            acc1 = (arows[r, pl.ds(L, L)] * brows[r, pl.ds(L, L)])
            for j in range(2, EMBED_SIZE // L, 2):
                acc0 = acc0 + arows[r, pl.ds(j * L, L)] * brows[r, pl.ds(j * L, L)]
                acc1 = acc1 + arows[r, pl.ds((j + 1) * L, L)] * brows[r, pl.ds((j + 1) * L, L)]
            v = acc0 + acc1
            for h in (8, 4, 2, 1):
                v = v + jnp.take(v, perms[h])
            plsc.store_scatter(dots, [jnp.full((L,), c * CHUNK + r, jnp.int32)],
                               v, mask=mask0)

    cp_sg.wait()

    @plsc.parallel_loop(0, B_PER_W // L, unroll=4)
    def _loss_body(i):
        off = pl.multiple_of(i * L, L)
        t = sign_v[pl.ds(off, L)] * dots[pl.ds(off, L)]
        u = jnp.exp(-jnp.abs(t))
        dots[pl.ds(off, L)] = jnp.maximum(-t, 0.0) + _log1p_poly(u)

    pltpu.sync_copy(dots, out_hbm.at[pl.ds(base, B_PER_W)])


@jax.jit
def _run(a2d, b2d, sign, emb_table, ctx_table):
    mesh = plsc.VectorSubcoreMesh(core_axis_name="c", subcore_axis_name="s")
    f = pl.kernel(
        _sc_kernel,
        mesh=mesh,
        compiler_params=pltpu.CompilerParams(needs_layout_passes=False),
        out_type=jax.ShapeDtypeStruct((BATCH,), jnp.float32),
        scratch_types=[
            pltpu.VMEM((NCHUNK, CHUNK), jnp.int32),
            pltpu.VMEM((NCHUNK, CHUNK), jnp.int32),
            pltpu.VMEM((B_PER_W,), jnp.float32),
            pltpu.VMEM((B_PER_W,), jnp.float32),
            [pltpu.VMEM((CHUNK, EMBED_SIZE), jnp.float32)
             for _ in range(2 * NBUF)],
            [pltpu.SemaphoreType.DMA for _ in range(2 * NBUF)],
            pltpu.SemaphoreType.DMA,
            pltpu.SemaphoreType.DMA,
        ],
    )
    return f(a2d, b2d, sign, emb_table, ctx_table)


def kernel(a, b, sign, emb_table, ctx_table):
    a2d = a.reshape(BATCH // CHUNK, CHUNK)
    b2d = b.reshape(BATCH // CHUNK, CHUNK)
    return _run(a2d, b2d, sign, emb_table, ctx_table)
